# baseline (speedup 1.0000x reference)
import functools

import jax
import jax.numpy as jnp
from jax import lax
from jax.experimental import pallas as pl
from jax.experimental.pallas import tpu as pltpu
from jax.experimental.pallas import tpu_sc as plsc

ROWS = 1_000_000
COLS = 64
B = 16384
TOTAL = B * COLS
NC, NS, L = 2, 16, 16
NW = NC * NS
PER_W = TOTAL // NW
CHUNK = 128
NCHUNK = PER_W // CHUNK

_mesh = plsc.VectorSubcoreMesh(core_axis_name="c", subcore_axis_name="s")


@functools.partial(
    pl.kernel,
    mesh=_mesh,
    out_type=jax.ShapeDtypeStruct((TOTAL,), jnp.float32),
    scratch_types=[
        pltpu.VMEM((PER_W,), jnp.int32),
        pltpu.VMEM((PER_W,), jnp.float32),
        pltpu.SemaphoreType.DMA,
    ],
)
def _sc_gather(inp_hbm, idx_hbm, out_hbm, idx_v, out_v, sem):
    wid = lax.axis_index("s") * NC + lax.axis_index("c")
    base = wid * PER_W

    pltpu.sync_copy(idx_hbm.at[pl.ds(base, PER_W)], idx_v)

    lane = lax.iota(jnp.int32, L)

    def addr_body(q, _):
        for r in range(4):
            sl = pl.ds(q * (4 * L) + r * L, L)
            idx_v[sl] = idx_v[sl] * COLS + (lane + r * L)
        return 0

    lax.fori_loop(0, PER_W // (4 * L), addr_body, 0)

    def gather_body(c, _):
        cp = pltpu.make_async_copy(
            inp_hbm.at[idx_v.at[pl.ds(c * CHUNK, CHUNK)]],
            out_v.at[pl.ds(c * CHUNK, CHUNK)],
            sem,
        )
        cp.start()
        cp.wait()
        return 0

    lax.fori_loop(0, NCHUNK, gather_body, 0)

    pltpu.sync_copy(out_v, out_hbm.at[pl.ds(base, PER_W)])


def kernel(input, index):
    inp_flat = input.reshape(ROWS * COLS)
    idx_flat = index.reshape(TOTAL).astype(jnp.int32)
    out_flat = _sc_gather(inp_flat, idx_flat)
    return out_flat.reshape(B, COLS)


# speedup vs baseline: 1.2238x; 1.2238x over previous
import functools

import jax
import jax.numpy as jnp
from jax import lax
from jax.experimental import pallas as pl
from jax.experimental.pallas import tpu as pltpu
from jax.experimental.pallas import tpu_sc as plsc

ROWS = 1_000_000
COLS = 64
B = 16384
TOTAL = B * COLS
NC, NS, L = 2, 16, 16
NW = NC * NS
PER_W = TOTAL // NW
CHUNK = 128
NCHUNK = PER_W // CHUNK

_mesh = plsc.VectorSubcoreMesh(core_axis_name="c", subcore_axis_name="s")


@functools.partial(
    pl.kernel,
    mesh=_mesh,
    out_type=jax.ShapeDtypeStruct((TOTAL,), jnp.float32),
    scratch_types=[
        pltpu.VMEM((PER_W,), jnp.int32),
        pltpu.VMEM((PER_W,), jnp.float32),
        pltpu.SemaphoreType.DMA,
    ],
)
def _sc_gather(inp_hbm, idx_hbm, out_hbm, idx_v, out_v, sem):
    wid = lax.axis_index("s") * NC + lax.axis_index("c")
    base = wid * PER_W

    pltpu.sync_copy(idx_hbm.at[pl.ds(base, PER_W)], idx_v)

    lane = lax.iota(jnp.int32, L)

    def addr_body(q, _):
        for r in range(4):
            sl = pl.ds(q * (4 * L) + r * L, L)
            idx_v[sl] = idx_v[sl] * COLS + (lane + r * L)
        return 0

    lax.fori_loop(0, PER_W // (4 * L), addr_body, 0)

    def _copy(c):
        return pltpu.make_async_copy(
            inp_hbm.at[idx_v.at[pl.ds(c * CHUNK, CHUNK)]],
            out_v.at[pl.ds(c * CHUNK, CHUNK)],
            sem,
        )

    def fire_body(c, _):
        _copy(c).start()
        return 0

    lax.fori_loop(0, NCHUNK, fire_body, 0)

    def drain_body(c, _):
        _copy(c).wait()
        return 0

    lax.fori_loop(0, NCHUNK, drain_body, 0)

    pltpu.sync_copy(out_v, out_hbm.at[pl.ds(base, PER_W)])


def kernel(input, index):
    inp_flat = input.reshape(ROWS * COLS)
    idx_flat = index.reshape(TOTAL).astype(jnp.int32)
    out_flat = _sc_gather(inp_flat, idx_flat)
    return out_flat.reshape(B, COLS)
